# baseline (speedup 1.0000x reference)
import functools

import jax
import jax.numpy as jnp
from jax import lax
from jax.experimental import pallas as pl
from jax.experimental.pallas import tpu as pltpu, tpu_sc as plsc

N = 10000
NP = 10240
E = 320000
D = 128
NC, NS = 2, 16
NW = NC * NS
EPW = E // NW
C = 96
KC = 112
EPWP = KC * C
KB = 16
NB = KC // KB
RPT = NP // NS

_MESH = plsc.VectorSubcoreMesh(core_axis_name="c", subcore_axis_name="s",
                               num_cores=NC, num_subcores=NS)


def _sc_body(compute_cnt, x_hbm, src_hbm, dst_hbm, *rest):
    if compute_cnt:
        (sum_out, cnt_out, src_v, dst_v, rows_v, rows2_v, rows3_v, zcnt_v,
         ones_v, sem, sem2, sem3, semc, acc_sh, cnt_sh) = rest
    else:
        (sum_out, src_v, dst_v, rows_v, rows2_v, rows3_v, sem, sem2, sem3,
         acc_sh) = rest
    bufs = (rows_v, rows2_v, rows3_v)
    sems = (sem, sem2, sem3)

    c = lax.axis_index("c")
    s = lax.axis_index("s")
    w = c * NS + s
    base_r = s * RPT

    def _zrow(i, carry):
        for k in range(D // 16):
            rows_v[i, pl.ds(k * 16, 16)] = jnp.zeros((16,), jnp.float32)
        return carry

    lax.fori_loop(0, C, _zrow, 0)
    zfull, zrem = divmod(RPT, C)
    for k in range(zfull):
        pltpu.sync_copy(rows_v, acc_sh.at[pl.ds(base_r + k * C, C)])
    if zrem:
        pltpu.sync_copy(rows_v.at[pl.ds(0, zrem)],
                        acc_sh.at[pl.ds(base_r + zfull * C, zrem)])

    if compute_cnt:
        def _zcnt(i, carry):
            zcnt_v[pl.ds(i * 16, 16)] = jnp.zeros((16,), jnp.float32)
            return carry

        lax.fori_loop(0, RPT // 16, _zcnt, 0)
        pltpu.sync_copy(zcnt_v, cnt_sh.at[pl.ds(base_r, RPT)])
        for k in range(C // 16):
            ones_v[pl.ds(k * 16, 16)] = jnp.ones((16,), jnp.float32)

    def _consume(j, buf):
        if compute_cnt:
            pltpu.async_copy(ones_v, cnt_sh.at[dst_v.at[j]], semc, add=True)
        pltpu.sync_copy(buf, acc_sh.at[dst_v.at[j]], add=True)
        if compute_cnt:
            pltpu.make_async_copy(ones_v, cnt_sh.at[dst_v.at[0]], semc).wait()

    def _drain(buf, sm):
        pltpu.make_async_copy(x_hbm.at[src_v.at[0]], buf, sm).wait()

    def _block(kb, carry):
        pltpu.sync_copy(src_hbm.at[w, pl.ds(kb * KB, KB)], src_v)
        pltpu.sync_copy(dst_hbm.at[w, pl.ds(kb * KB, KB)], dst_v)
        pltpu.async_copy(x_hbm.at[src_v.at[0]], bufs[0], sems[0])
        pltpu.async_copy(x_hbm.at[src_v.at[1]], bufs[1], sems[1])
        for j in range(KB):
            if j + 2 < KB:
                pltpu.async_copy(x_hbm.at[src_v.at[j + 2]],
                                 bufs[(j + 2) % 3], sems[(j + 2) % 3])
            _drain(bufs[j % 3], sems[j % 3])
            _consume(j, bufs[j % 3])
        return carry

    plsc.subcore_barrier()
    lax.fori_loop(0, NB, _block, 0)
    plsc.subcore_barrier()

    pltpu.sync_copy(acc_sh.at[pl.ds(base_r, RPT)],
                    sum_out.at[c, pl.ds(base_r, RPT)])
    if compute_cnt:
        pltpu.sync_copy(cnt_sh.at[pl.ds(base_r, RPT)],
                        cnt_out.at[c, pl.ds(base_r, RPT)])


def _make_sc_agg(compute_cnt):
    out_type = [jax.ShapeDtypeStruct((NC, NP, D), jnp.float32)]
    scratch = [
        pltpu.VMEM((KB, C), jnp.int32),
        pltpu.VMEM((KB, C), jnp.int32),
        pltpu.VMEM((C, D), jnp.float32),
        pltpu.VMEM((C, D), jnp.float32),
        pltpu.VMEM((C, D), jnp.float32),
    ]
    if compute_cnt:
        out_type.append(jax.ShapeDtypeStruct((NC, NP), jnp.float32))
        scratch.append(pltpu.VMEM((RPT,), jnp.float32))
        scratch.append(pltpu.VMEM((C,), jnp.float32))
    scratch.append(pltpu.SemaphoreType.DMA)
    scratch.append(pltpu.SemaphoreType.DMA)
    scratch.append(pltpu.SemaphoreType.DMA)
    if compute_cnt:
        scratch.append(pltpu.SemaphoreType.DMA)
    scratch.append(pltpu.VMEM_SHARED((NP, D), jnp.float32))
    if compute_cnt:
        scratch.append(pltpu.VMEM_SHARED((NP,), jnp.float32))

    return pl.kernel(
        functools.partial(_sc_body, compute_cnt),
        out_type=tuple(out_type),
        mesh=_MESH,
        scratch_types=scratch,
    )


_sc_agg_cnt = _make_sc_agg(True)
_sc_agg = _make_sc_agg(False)

BT = 1280


def _tc_body(sp_ref, cp_ref, x_ref, wl_ref, wr_ref, b_ref, o_ref):
    summed = sp_ref[0] + sp_ref[1]
    cnt = cp_ref[0] + cp_ref[1]
    mean = summed * (1.0 / jnp.maximum(cnt, 1.0))[:, None]
    out = (jnp.dot(mean, wl_ref[...], preferred_element_type=jnp.float32)
           + jnp.dot(x_ref[...], wr_ref[...], preferred_element_type=jnp.float32)
           + b_ref[...])
    o_ref[...] = jnp.maximum(out, 0.0)


def _tc_layer(sp, cp, x, W_l, W_r, b, rows_out):
    return pl.pallas_call(
        _tc_body,
        grid=(NP // BT,),
        in_specs=[
            pl.BlockSpec((NC, BT, D), lambda i: (0, i, 0)),
            pl.BlockSpec((NC, BT), lambda i: (0, i)),
            pl.BlockSpec((BT, D), lambda i: (i, 0)),
            pl.BlockSpec((D, D), lambda i: (0, 0)),
            pl.BlockSpec((D, D), lambda i: (0, 0)),
            pl.BlockSpec((1, D), lambda i: (0, 0)),
        ],
        out_specs=pl.BlockSpec((BT, D), lambda i: (i, 0)),
        out_shape=jax.ShapeDtypeStruct((rows_out, D), jnp.float32),
    )(sp, cp, x, W_l, W_r, b)


def kernel(x, W0_l, b0_l, W0_r, b0_r, W1_l, b1_l, W1_r, b1_r, edge_index):
    pad = EPWP - EPW
    dummy_src = jnp.broadcast_to(jnp.arange(pad, dtype=jnp.int32) % N,
                                 (NW, pad))
    dummy_dst = jnp.broadcast_to(N + (jnp.arange(pad, dtype=jnp.int32)
                                      % (NP - N)), (NW, pad))
    src3d = jnp.concatenate([edge_index[0].reshape(NW, EPW), dummy_src],
                            axis=1).reshape(NW, KC, C)
    dst3d = jnp.concatenate([edge_index[1].reshape(NW, EPW), dummy_dst],
                            axis=1).reshape(NW, KC, C)

    sp1, cp = _sc_agg_cnt(x, src3d, dst3d)
    z1 = _tc_layer(sp1, cp, x, W0_l, W0_r, (b0_l + b0_r).reshape(1, D), NP)

    (sp2,) = _sc_agg(z1, src3d, dst3d)
    return _tc_layer(sp2, cp, z1, W1_l, W1_r, (b1_l + b1_r).reshape(1, D), N)


# speedup vs baseline: 1.0675x; 1.0675x over previous
import functools

import jax
import jax.numpy as jnp
from jax import lax
from jax.experimental import pallas as pl
from jax.experimental.pallas import tpu as pltpu, tpu_sc as plsc

N = 10000
NP = 10240
E = 320000
D = 128
NC, NS = 2, 16
NW = NC * NS
EPW = E // NW
C = 128
KC = 80
EPWP = KC * C
KB = 40
NB = KC // KB
RPT = NP // NS

_MESH = plsc.VectorSubcoreMesh(core_axis_name="c", subcore_axis_name="s",
                               num_cores=NC, num_subcores=NS)


def _sc_body(compute_cnt, x_hbm, src_hbm, dst_hbm, *rest):
    if compute_cnt:
        (sum_out, cnt_out, src_v, dst_v, rows_v, rows2_v, zcnt_v, ones_v,
         sem, sem2, sem3, acc_sh, cnt_sh) = rest
    else:
        (sum_out, src_v, dst_v, rows_v, rows2_v, sem, sem2, acc_sh) = rest

    c = lax.axis_index("c")
    s = lax.axis_index("s")
    w = c * NS + s
    base_r = s * RPT

    def _zrow(i, carry):
        for k in range(D // 16):
            rows_v[i, pl.ds(k * 16, 16)] = jnp.zeros((16,), jnp.float32)
        return carry

    lax.fori_loop(0, C, _zrow, 0)
    for k in range(RPT // C):
        pltpu.sync_copy(rows_v, acc_sh.at[pl.ds(base_r + k * C, C)])

    if compute_cnt:
        def _zcnt(i, carry):
            zcnt_v[pl.ds(i * 16, 16)] = jnp.zeros((16,), jnp.float32)
            return carry

        lax.fori_loop(0, RPT // 16, _zcnt, 0)
        pltpu.sync_copy(zcnt_v, cnt_sh.at[pl.ds(base_r, RPT)])
        for k in range(C // 16):
            ones_v[pl.ds(k * 16, 16)] = jnp.ones((16,), jnp.float32)

    def _consume(j, buf):
        if compute_cnt:
            pltpu.async_copy(ones_v, cnt_sh.at[dst_v.at[j]], sem3, add=True)
        pltpu.sync_copy(buf, acc_sh.at[dst_v.at[j]], add=True)
        if compute_cnt:
            pltpu.make_async_copy(ones_v, cnt_sh.at[dst_v.at[0]], sem3).wait()

    def _drain(buf, sm):
        pltpu.make_async_copy(x_hbm.at[src_v.at[0]], buf, sm).wait()

    def _stage_idx(kb):
        pltpu.sync_copy(src_hbm.at[w, pl.ds(kb * KB, KB)], src_v)
        pltpu.sync_copy(dst_hbm.at[w, pl.ds(kb * KB, KB)], dst_v)

    def _pipe():
        def _step(i, carry2):
            j0 = 2 * i
            pltpu.async_copy(x_hbm.at[src_v.at[j0 + 1]], rows2_v, sem2)
            _drain(rows_v, sem)
            _consume(j0, rows_v)

            @pl.when(i < KB // 2 - 1)
            def _():
                pltpu.async_copy(x_hbm.at[src_v.at[j0 + 2]], rows_v, sem)

            _drain(rows2_v, sem2)
            _consume(j0 + 1, rows2_v)
            return carry2

        lax.fori_loop(0, KB // 2, _step, 0)

    _stage_idx(0)
    pltpu.async_copy(x_hbm.at[src_v.at[0]], rows_v, sem)
    plsc.subcore_barrier()
    _pipe()
    for kb in range(1, NB):
        _stage_idx(kb)
        pltpu.async_copy(x_hbm.at[src_v.at[0]], rows_v, sem)
        _pipe()

    plsc.subcore_barrier()

    pltpu.sync_copy(acc_sh.at[pl.ds(base_r, RPT)],
                    sum_out.at[c, pl.ds(base_r, RPT)])
    if compute_cnt:
        pltpu.sync_copy(cnt_sh.at[pl.ds(base_r, RPT)],
                        cnt_out.at[c, pl.ds(base_r, RPT)])


def _make_sc_agg(compute_cnt):
    out_type = [jax.ShapeDtypeStruct((NC, NP, D), jnp.float32)]
    scratch = [
        pltpu.VMEM((KB, C), jnp.int32),
        pltpu.VMEM((KB, C), jnp.int32),
        pltpu.VMEM((C, D), jnp.float32),
        pltpu.VMEM((C, D), jnp.float32),
    ]
    if compute_cnt:
        out_type.append(jax.ShapeDtypeStruct((NC, NP), jnp.float32))
        scratch.append(pltpu.VMEM((RPT,), jnp.float32))
        scratch.append(pltpu.VMEM((C,), jnp.float32))
    scratch.append(pltpu.SemaphoreType.DMA)
    scratch.append(pltpu.SemaphoreType.DMA)
    if compute_cnt:
        scratch.append(pltpu.SemaphoreType.DMA)
    scratch.append(pltpu.VMEM_SHARED((NP, D), jnp.float32))
    if compute_cnt:
        scratch.append(pltpu.VMEM_SHARED((NP,), jnp.float32))

    return pl.kernel(
        functools.partial(_sc_body, compute_cnt),
        out_type=tuple(out_type),
        mesh=_MESH,
        scratch_types=scratch,
    )


_sc_agg_cnt = _make_sc_agg(True)
_sc_agg = _make_sc_agg(False)

BT = 1280


def _tc_body(sp_ref, cp_ref, x_ref, wl_ref, wr_ref, b_ref, o_ref):
    summed = sp_ref[0] + sp_ref[1]
    cnt = cp_ref[0] + cp_ref[1]
    mean = summed * (1.0 / jnp.maximum(cnt, 1.0))[:, None]
    out = (jnp.dot(mean, wl_ref[...], preferred_element_type=jnp.float32)
           + jnp.dot(x_ref[...], wr_ref[...], preferred_element_type=jnp.float32)
           + b_ref[...])
    o_ref[...] = jnp.maximum(out, 0.0)


def _tc_layer(sp, cp, x, W_l, W_r, b, rows_out):
    return pl.pallas_call(
        _tc_body,
        grid=(NP // BT,),
        in_specs=[
            pl.BlockSpec((NC, BT, D), lambda i: (0, i, 0)),
            pl.BlockSpec((NC, BT), lambda i: (0, i)),
            pl.BlockSpec((BT, D), lambda i: (i, 0)),
            pl.BlockSpec((D, D), lambda i: (0, 0)),
            pl.BlockSpec((D, D), lambda i: (0, 0)),
            pl.BlockSpec((1, D), lambda i: (0, 0)),
        ],
        out_specs=pl.BlockSpec((BT, D), lambda i: (i, 0)),
        out_shape=jax.ShapeDtypeStruct((rows_out, D), jnp.float32),
    )(sp, cp, x, W_l, W_r, b)


def kernel(x, W0_l, b0_l, W0_r, b0_r, W1_l, b1_l, W1_r, b1_r, edge_index):
    pad = EPWP - EPW
    dummy_src = jnp.broadcast_to(jnp.arange(pad, dtype=jnp.int32) % N,
                                 (NW, pad))
    dummy_dst = jnp.broadcast_to(N + (jnp.arange(pad, dtype=jnp.int32)
                                      % (NP - N)), (NW, pad))
    src3d = jnp.concatenate([edge_index[0].reshape(NW, EPW), dummy_src],
                            axis=1).reshape(NW, KC, C)
    dst3d = jnp.concatenate([edge_index[1].reshape(NW, EPW), dummy_dst],
                            axis=1).reshape(NW, KC, C)

    sp1, cp = _sc_agg_cnt(x, src3d, dst3d)
    z1 = _tc_layer(sp1, cp, x, W0_l, W0_r, (b0_l + b0_r).reshape(1, D), NP)

    (sp2,) = _sc_agg(z1, src3d, dst3d)
    return _tc_layer(sp2, cp, z1, W1_l, W1_r, (b1_l + b1_r).reshape(1, D), N)
